# baseline (speedup 1.0000x reference)
import jax
import jax.numpy as jnp
from jax import lax
from jax.experimental import pallas as pl
from jax.experimental.pallas import tpu as pltpu
from jax.experimental.pallas import tpu_sc as plsc

_VOCAB = 1000000
_D = 64
_BATCH = 16384
_HIST = 200

_NC = 2
_NS = 16
_NW = _NC * _NS
_BPW = _BATCH // _NW
_TC = 128
_NTC = _BPW // _TC
_TR = _HIST // 8
_NBLK = _NTC * _TR
_NH = 2 * _NBLK
_HPC = 2 * _TR
_NSTEP = _NH // 4
_HR = 512
_LN = _D // 16


def _body(xP_hbm, table_hbm, out_hbm, idx_v, rows_v, acc, out_s,
          si0, si1, sr0, sr1, so):
    wid = lax.axis_index("s") * _NC + lax.axis_index("c")
    base = wid * _BPW
    tc0 = wid * _NTC
    sems_i = (si0, si1)
    sems_r = (sr0, sr1)
    inv = jnp.float32(1.0 / _HIST)

    def blk_src(beta):
        tr = beta % _TR
        tci = tc0 + beta // _TR
        return xP_hbm.at[tr, tci]

    def idx_dma(beta, q):
        pltpu.async_copy(blk_src(beta), idx_v.at[q], sems_i[q])

    def idx_wait(beta, q):
        pltpu.make_async_copy(blk_src(beta), idx_v.at[q], sems_i[q]).wait()

    def gather(h, q, p):
        off = _HR * (h % 2)
        pltpu.async_copy(
            table_hbm.at[idx_v.at[q, pl.ds(off, _HR)]], rows_v.at[p],
            sems_r[p])

    def gather_wait(q, p):
        pltpu.make_async_copy(
            table_hbm.at[idx_v.at[q, pl.ds(0, _HR)]], rows_v.at[p],
            sems_r[p]).wait()

    def out_dst(h):
        return out_hbm.at[pl.ds(base + (h // _HPC) * _TC, _TC)]

    def out_write(h):
        pltpu.async_copy(out_s, out_dst(h), so)

    def out_wait(h):
        pltpu.make_async_copy(out_s, out_dst(h), so).wait()

    def zero_acc():
        z = jnp.zeros((16,), jnp.float32)

        def zb(s, carry):
            for t in range(_LN):
                acc[s, pl.ds(t * 16, 16)] = z
            return carry

        lax.fori_loop(0, _TC, zb, 0)

    def accumulate(p):
        def ab(s, carry):
            for t in range(_LN):
                a = acc[s, pl.ds(t * 16, 16)]
                for r in range(4):
                    a = a + rows_v[p, r * _TC + s, pl.ds(t * 16, 16)]
                acc[s, pl.ds(t * 16, 16)] = a
            return carry

        lax.fori_loop(0, _TC, ab, 0)

    def scale_out():
        def sb(s, carry):
            for t in range(_LN):
                out_s[s, pl.ds(t * 16, 16)] = acc[s, pl.ds(t * 16, 16)] * inv
            return carry

        lax.fori_loop(0, _TC, sb, 0)

    def slot(step, b):
        h = 4 * step + b
        p = b % 2
        q2 = (b // 2 + 1) % 2

        gather_wait(b // 2, p)

        @pl.when(h % _HPC == 0)
        def _():
            zero_acc()

        accumulate(p)

        @pl.when(jnp.logical_and(h % _HPC == _HPC - 1, h > _HPC))
        def _():
            out_wait(h - _HPC)

        @pl.when(h % _HPC == _HPC - 1)
        def _():
            scale_out()
            out_write(h)

        if b == 0:
            idx_wait(2 * step + 1, q2)
            gather(h + 2, q2, p)
        elif b == 1:
            gather(h + 2, q2, p)

            @pl.when(step < _NSTEP - 1)
            def _():
                idx_dma(2 * step + 2, 0)
        elif b == 2:
            @pl.when(step < _NSTEP - 1)
            def _():
                idx_wait(2 * step + 2, q2)
                gather(h + 2, q2, p)
        else:
            @pl.when(step < _NSTEP - 1)
            def _():
                gather(h + 2, q2, p)
                idx_dma(2 * step + 3, 1)

    idx_dma(0, 0)
    idx_dma(1, 1)
    idx_wait(0, 0)
    gather(0, 0, 0)
    gather(1, 0, 1)

    def step_body(step, carry):
        for b in range(4):
            slot(step, b)
        return carry

    lax.fori_loop(0, _NSTEP, step_body, 0)

    out_wait(_NH - 1)


@jax.jit
def kernel(x, table):
    xP = (x.T.astype(jnp.int32)
          .reshape(_TR, 8, _BATCH // _TC, _TC)
          .transpose(0, 2, 1, 3)
          .reshape(_TR, _BATCH // _TC, 8 * _TC))
    mesh = plsc.VectorSubcoreMesh(core_axis_name="c", subcore_axis_name="s")
    f = pl.kernel(
        _body,
        out_type=jax.ShapeDtypeStruct((_BATCH, _D), jnp.float32),
        mesh=mesh,
        scratch_types=[
            pltpu.VMEM((2, 8 * _TC), jnp.int32),
            pltpu.VMEM((2, _HR, _D), jnp.float32),
            pltpu.VMEM((_TC, _D), jnp.float32),
            pltpu.VMEM((_TC, _D), jnp.float32),
            pltpu.SemaphoreType.DMA,
            pltpu.SemaphoreType.DMA,
            pltpu.SemaphoreType.DMA,
            pltpu.SemaphoreType.DMA,
            pltpu.SemaphoreType.DMA,
        ],
        compiler_params=pltpu.CompilerParams(
            use_tc_tiling_on_sc=False, needs_layout_passes=False),
    )
    return f(xP, table)


# speedup vs baseline: 1.0422x; 1.0422x over previous
import jax
import jax.numpy as jnp
from jax import lax
from jax.experimental import pallas as pl
from jax.experimental.pallas import tpu as pltpu
from jax.experimental.pallas import tpu_sc as plsc

_VOCAB = 1000000
_D = 64
_BATCH = 16384
_HIST = 200

_NC = 2
_NS = 16
_NW = _NC * _NS
_BPW = _BATCH // _NW
_TC = 128
_NTC = _BPW // _TC
_TR = _HIST // 8
_NBLK = _NTC * _TR
_NH = 2 * _NBLK
_HPC = 2 * _TR
_NSTEP = _NH // 4
_HR = 512
_LN = _D // 16


def _body(xP_hbm, table_hbm, out_hbm, idx_v, rows_v, acc, out_s,
          si0, si1, sr0, sr1, so):
    wid = lax.axis_index("s") * _NC + lax.axis_index("c")
    base = wid * _BPW
    tc0 = wid * _NTC
    sems_i = (si0, si1)
    sems_r = (sr0, sr1)
    inv = jnp.float32(1.0 / _HIST)

    def blk_src(beta):
        tr = beta % _TR
        tci = tc0 + beta // _TR
        return xP_hbm.at[tr, tci]

    def idx_dma(beta, q):
        pltpu.async_copy(blk_src(beta), idx_v.at[q], sems_i[q])

    def idx_wait(beta, q):
        pltpu.make_async_copy(blk_src(beta), idx_v.at[q], sems_i[q]).wait()

    def double_idx(q):
        def db(i, carry):
            v = idx_v[q, pl.ds(i * 16, 16)]
            idx_v[q, pl.ds(i * 16, 16)] = v + v
            return carry

        lax.fori_loop(0, (8 * _TC) // 16, db, 0)

    def gather(h, q, p):
        off = _HR * (h % 2)
        pltpu.async_copy(
            table_hbm.at[idx_v.at[q, pl.ds(off, _HR)]], rows_v.at[p],
            sems_r[p])

    def gather_wait(q, p):
        pltpu.make_async_copy(
            table_hbm.at[idx_v.at[q, pl.ds(0, _HR)]], rows_v.at[p],
            sems_r[p]).wait()

    def out_dst(h):
        return out_hbm.at[pl.ds(base + (h // _HPC) * _TC, _TC)]

    def out_write(h):
        pltpu.async_copy(out_s, out_dst(h), so)

    def out_wait(h):
        pltpu.make_async_copy(out_s, out_dst(h), so).wait()

    def zero_acc():
        z = jnp.zeros((16,), jnp.float32)

        def zb(s, carry):
            for t in range(_LN):
                acc[s, pl.ds(t * 16, 16)] = z
            return carry

        lax.fori_loop(0, _TC, zb, 0)

    def accumulate(p):
        def ab(s, carry):
            for t in range(_LN):
                a = acc[s, pl.ds(t * 16, 16)]
                for r in range(4):
                    a = a + rows_v[p, r * _TC + s, pl.ds(t * 16, 16)]
                acc[s, pl.ds(t * 16, 16)] = a
            return carry

        lax.fori_loop(0, _TC, ab, 0)

    def scale_out():
        def sb(s, carry):
            for t in range(_LN):
                out_s[s, pl.ds(t * 16, 16)] = acc[s, pl.ds(t * 16, 16)] * inv
            return carry

        lax.fori_loop(0, _TC, sb, 0)

    def slot(step, b):
        h = 4 * step + b
        p = b % 2
        q2 = (b // 2 + 1) % 2

        gather_wait(b // 2, p)

        @pl.when(h % _HPC == 0)
        def _():
            zero_acc()

        accumulate(p)

        @pl.when(jnp.logical_and(h % _HPC == _HPC - 1, h > _HPC))
        def _():
            out_wait(h - _HPC)

        @pl.when(h % _HPC == _HPC - 1)
        def _():
            scale_out()
            out_write(h)

        if b == 0:
            idx_wait(2 * step + 1, q2)
            double_idx(q2)
            gather(h + 2, q2, p)
        elif b == 1:
            gather(h + 2, q2, p)

            @pl.when(step < _NSTEP - 1)
            def _():
                idx_dma(2 * step + 2, 0)
        elif b == 2:
            @pl.when(step < _NSTEP - 1)
            def _():
                idx_wait(2 * step + 2, q2)
                double_idx(q2)
                gather(h + 2, q2, p)
        else:
            @pl.when(step < _NSTEP - 1)
            def _():
                gather(h + 2, q2, p)
                idx_dma(2 * step + 3, 1)

    idx_dma(0, 0)
    idx_dma(1, 1)
    idx_wait(0, 0)
    double_idx(0)
    gather(0, 0, 0)
    gather(1, 0, 1)

    def step_body(step, carry):
        for b in range(4):
            slot(step, b)
        return carry

    lax.fori_loop(0, _NSTEP, step_body, 0)

    out_wait(_NH - 1)


@jax.jit
def kernel(x, table):
    xP = (x.T.astype(jnp.int32)
          .reshape(_TR, 8, _BATCH // _TC, _TC)
          .transpose(0, 2, 1, 3)
          .reshape(_TR, _BATCH // _TC, 8 * _TC))
    tableP = jnp.pad(table, ((0, 0), (0, 64))).reshape(2 * _VOCAB, _D)
    mesh = plsc.VectorSubcoreMesh(core_axis_name="c", subcore_axis_name="s")
    f = pl.kernel(
        _body,
        out_type=jax.ShapeDtypeStruct((_BATCH, _D), jnp.float32),
        mesh=mesh,
        scratch_types=[
            pltpu.VMEM((2, 8 * _TC), jnp.int32),
            pltpu.VMEM((2, _HR, _D), jnp.float32),
            pltpu.VMEM((_TC, _D), jnp.float32),
            pltpu.VMEM((_TC, _D), jnp.float32),
            pltpu.SemaphoreType.DMA,
            pltpu.SemaphoreType.DMA,
            pltpu.SemaphoreType.DMA,
            pltpu.SemaphoreType.DMA,
            pltpu.SemaphoreType.DMA,
        ],
        compiler_params=pltpu.CompilerParams(
            use_tc_tiling_on_sc=False, needs_layout_passes=False),
    )
    return f(xP, tableP)
